# baseline (speedup 1.0000x reference)
import jax
import jax.numpy as jnp
import numpy as np
from jax.experimental import pallas as pl
from jax.experimental.pallas import tpu as pltpu

H = 384
W = 384
CIN = 96
NC = 19
C0 = CIN + NC
O0 = C0 // 2
U0 = C0 * 2
O1 = O0 // 2
U1 = O0 * 2
GATE = 0.4

RSF = 8
NG = H // RSF
HB = 8
NHB = H // HB
NWC = W // 128
RH = RSF // HB


def _relu6(x):
    return jnp.clip(x, 0.0, 6.0)


def _mm(a, b):
    return jax.lax.dot_general(a.astype(jnp.bfloat16), b.astype(jnp.bfloat16),
                               (((1,), (0,)), ((), ())),
                               preferred_element_type=jnp.float32)


def _shift_w(x, s):
    if s == 0:
        return x
    z = jnp.zeros(x.shape[:-1] + (abs(s),), x.dtype)
    if s > 0:
        return jnp.concatenate([x[..., s:], z], axis=-1)
    return jnp.concatenate([z, x[..., :s]], axis=-1)


def _shift_w_fill(x, s, fill):
    if s == 0:
        return x
    z = jnp.full(x.shape[:-1] + (abs(s),), fill, x.dtype)
    if s > 0:
        return jnp.concatenate([x[..., s:], z], axis=-1)
    return jnp.concatenate([z, x[..., :s]], axis=-1)


def _dw3x3(x, w9):
    C, R, Wd = x.shape
    acc = jnp.zeros((C, R - 2, Wd), jnp.float32)
    for ky in range(3):
        xs = x[:, ky:ky + R - 2, :]
        for kx in range(3):
            coef = w9[:, 3 * ky + kx].reshape(C, 1, 1)
            acc = acc + coef * _shift_w(xs, kx - 1)
    return acc


def _pw_bn_relu(x3, pw, sc, bi):
    C, R, Wd = x3.shape
    y = _mm(pw, x3.reshape(C, R * Wd))
    y = jnp.maximum(y * sc + bi, 0.0)
    return y.reshape(pw.shape[0], R, Wd)


def _sep(x3, w9, pw, sc, bi):
    return _pw_bn_relu(_dw3x3(x3, w9), pw, sc, bi)


def _zero_outside(x, base):
    r = jax.lax.broadcasted_iota(jnp.int32, (1, x.shape[1], 1), 1) + base
    return jnp.where((r >= 0) & (r < H), x, 0.0)


def _mlp_blend(x2, mb, wi, bi, wm0, bm0, wm1, bm1, wo, bo):
    ur = _relu6(_mm(wi, x2) + bi)
    ur = ur + _relu6(_mm(wm0, ur) + bm0)
    ur = ur + _relu6(_mm(wm1, ur) + bm1)
    ur = _relu6(_mm(wo, ur) + bo)
    return jnp.where(mb > 0.0, ur, x2)


def _row_chunks(n):
    out = []
    r = 0
    while r < n:
        out.append((r, min(r + 8, n)))
        r += 8
    return out


def _mlp_pass(src, scratch_ref, mask, mw):
    C, R, Wd = src.shape
    scratch_ref[...] = src
    for (r0, r1) in _row_chunks(R):
        for wc in range(NWC):
            sl = slice(wc * 128, (wc + 1) * 128)
            xm = src[:, r0:r1, sl].reshape(C, (r1 - r0) * 128)
            mb = mask[r0:r1, sl].reshape(1, (r1 - r0) * 128)

            @pl.when(jnp.max(mb) > 0.0)
            def _(xm=xm, mb=mb, r0=r0, r1=r1, sl=sl):
                res = _mlp_blend(xm, mb, *mw)
                scratch_ref[:, r0:r1, sl] = res.reshape(C, r1 - r0, 128)
    return scratch_ref[...]


def _fused_body(fu_ref, fm_ref, fd_ref, cu_ref, cm_ref, cd_ref, *rest):
    (m0wi, m0bi, m0wm0, m0bm0, m0wm1, m0bm1, m0wo, m0bo,
     c0w90, c0pw0, c0sc0, c0bi0, c0w91, c0pw1, c0sc1, c0bi1,
     c0w9o, c0pwo, c0sco, c0bio,
     m1wi, m1bi, m1wm0, m1bm0, m1wm1, m1bm1, m1wo, m1bo,
     c1w90, c1pw0, c1sc0, c1bi0, c1w91, c1pw1, c1sc1, c1bi1,
     c1w9o, c1pwo, c1sco, c1bio,
     out_ref, x0_s, x1_s) = rest
    j = pl.program_id(0)
    g0 = j * RSF

    featw = jnp.concatenate([fu_ref[...], fm_ref[...], fd_ref[...]], axis=1)
    cpw = jnp.concatenate([cu_ref[...], cm_ref[...], cd_ref[...]], axis=1)

    unc = 1.0 - jnp.max(cpw[:, HB - 7:HB + RSF + 7, :], axis=0)
    runc = jax.lax.broadcasted_iota(jnp.int32, (RSF + 14, 1), 0) + (g0 - 7)
    unc = jnp.where((runc >= 0) & (runc < H), unc, -1.0)
    mv = jnp.maximum(jnp.maximum(unc[:-2], unc[1:-1]), unc[2:])
    dil = jnp.maximum(jnp.maximum(_shift_w_fill(mv, -1, -1.0),
                                  _shift_w_fill(mv, 1, -1.0)), mv)
    mask44 = (dil > GATE).astype(jnp.float32)

    x0 = jnp.concatenate([featw[:, HB - 6:HB + RSF + 6, :],
                          cpw[:, HB - 6:HB + RSF + 6, :]], axis=0)
    x0 = _zero_outside(x0, g0 - 6)
    m0w = (m0wi[...], m0bi[...], m0wm0[...], m0bm0[...],
           m0wm1[...], m0bm1[...], m0wo[...], m0bo[...])
    x0w = _mlp_pass(x0, x0_s, mask44, m0w)

    y1 = x0w[:, 1:RSF + 11, :] + _sep(x0w, c0w90[...], c0pw0[...],
                                      c0sc0[...], c0bi0[...])
    y1 = _zero_outside(y1, g0 - 5)
    y2 = y1[:, 1:RSF + 9, :] + _sep(y1, c0w91[...], c0pw1[...],
                                    c0sc1[...], c0bi1[...])
    y2 = _zero_outside(y2, g0 - 4)
    o1 = _sep(y2, c0w9o[...], c0pwo[...], c0sco[...], c0bio[...])
    o1 = _zero_outside(o1, g0 - 3)

    m1w = (m1wi[...], m1bi[...], m1wm0[...], m1bm0[...],
           m1wm1[...], m1bm1[...], m1wo[...], m1bo[...])
    x1w = _mlp_pass(o1, x1_s, mask44[3:RSF + 9], m1w)

    z1 = x1w[:, 1:RSF + 5, :] + _sep(x1w, c1w90[...], c1pw0[...],
                                     c1sc0[...], c1bi0[...])
    z1 = _zero_outside(z1, g0 - 2)
    z2 = z1[:, 1:RSF + 3, :] + _sep(z1, c1w91[...], c1pw1[...],
                                    c1sc1[...], c1bi1[...])
    z2 = _zero_outside(z2, g0 - 1)
    out_ref[...] = _sep(z2, c1w9o[...], c1pwo[...], c1sco[...], c1bio[...])


def _mlp_weights(p, b):
    return (p[f'b{b}_win'], p[f'b{b}_bin'].reshape(-1, 1),
            p[f'b{b}_wm0'], p[f'b{b}_bm0'].reshape(-1, 1),
            p[f'b{b}_wm1'], p[f'b{b}_bm1'].reshape(-1, 1),
            p[f'b{b}_wout'], p[f'b{b}_bout'].reshape(-1, 1))


def _conv_weights(p, b):
    out = []
    scale = np.float32(1.0 / np.sqrt(1.0 + 1e-5))
    for tag in ('0', '1', 'o'):
        dw = p[f'b{b}_dw{tag}']
        pw = p[f'b{b}_pw{tag}']
        g = p[f'b{b}_g{tag}']
        be = p[f'b{b}_be{tag}']
        out += [dw.reshape(dw.shape[0], 9),
                pw.reshape(pw.shape[0], pw.shape[1]),
                (g * scale).reshape(-1, 1), be.reshape(-1, 1)]
    return tuple(out)


def _cl(i):
    return jnp.clip(i, 0, NHB - 1)


def kernel(feature_map, coarse_pred, params):
    feat = feature_map[0]
    cp = coarse_pred[0]
    p = params
    ws = (_mlp_weights(p, 0) + _conv_weights(p, 0)
          + _mlp_weights(p, 1) + _conv_weights(p, 1))
    full = lambda shp: pl.BlockSpec(shp, lambda j: (0,) * len(shp))
    out = pl.pallas_call(
        _fused_body,
        grid=(NG,),
        out_shape=jax.ShapeDtypeStruct((O1, H, W), jnp.float32),
        in_specs=[
            pl.BlockSpec((CIN, HB, W), lambda j: (0, _cl(RH * j - 1), 0)),
            pl.BlockSpec((CIN, RSF, W), lambda j: (0, j, 0)),
            pl.BlockSpec((CIN, HB, W), lambda j: (0, _cl(RH * (j + 1)), 0)),
            pl.BlockSpec((NC, HB, W), lambda j: (0, _cl(RH * j - 1), 0)),
            pl.BlockSpec((NC, RSF, W), lambda j: (0, j, 0)),
            pl.BlockSpec((NC, HB, W), lambda j: (0, _cl(RH * (j + 1)), 0)),
        ] + [full(w.shape) for w in ws],
        out_specs=pl.BlockSpec((O1, RSF, W), lambda j: (0, j, 0)),
        scratch_shapes=[
            pltpu.VMEM((C0, RSF + 12, W), jnp.float32),
            pltpu.VMEM((O0, RSF + 6, W), jnp.float32),
        ],
        compiler_params=pltpu.CompilerParams(
            dimension_semantics=("arbitrary",),
            vmem_limit_bytes=56 * 1024 * 1024),
    )(feat, feat, feat, cp, cp, cp, *ws)
    return out[None]


# speedup vs baseline: 1.9440x; 1.9440x over previous
import jax
import jax.numpy as jnp
import numpy as np
from jax.experimental import pallas as pl
from jax.experimental.pallas import tpu as pltpu

H = 384
W = 384
CIN = 96
NC = 19
C0 = CIN + NC
O0 = C0 // 2
U0 = C0 * 2
O1 = O0 // 2
U1 = O0 * 2
GATE = 0.4

RS = 8
PAD = 16
HP = H + 2 * PAD
NGM = HP // RS
RSC = 16
NGC = HP // RSC
NB8 = HP // 8
NWC = W // 128


def _relu6(x):
    return jnp.clip(x, 0.0, 6.0)


def _mm(a, b):
    return jax.lax.dot_general(a, b, (((1,), (0,)), ((), ())),
                               preferred_element_type=jnp.float32)


def _shift_w(x, s):
    if s == 0:
        return x
    z = jnp.zeros(x.shape[:-1] + (abs(s),), x.dtype)
    if s > 0:
        return jnp.concatenate([x[..., s:], z], axis=-1)
    return jnp.concatenate([z, x[..., :s]], axis=-1)


def _dw3x3(x, w9):
    C, R, Wd = x.shape
    acc = jnp.zeros((C, R - 2, Wd), jnp.float32)
    for ky in range(3):
        xs = x[:, ky:ky + R - 2, :]
        for kx in range(3):
            coef = w9[:, 3 * ky + kx].reshape(C, 1, 1)
            acc = acc + coef * _shift_w(xs, kx - 1)
    return acc


def _pw_bn_relu(x3, pw, sc, bi):
    C, R, Wd = x3.shape
    y = _mm(pw, x3.reshape(C, R * Wd))
    y = jnp.maximum(y * sc + bi, 0.0)
    return y.reshape(pw.shape[0], R, Wd)


def _sep(x3, w9, pw, sc, bi):
    return _pw_bn_relu(_dw3x3(x3, w9), pw, sc, bi)


def _zero_outside(x, base):
    r = jax.lax.broadcasted_iota(jnp.int32, (1, x.shape[1], 1), 1) + base
    return jnp.where((r >= 0) & (r < H), x, 0.0)


def _mlp_blend(x2, mb, wi, bi, wm0, bm0, wm1, bm1, wo, bo):
    ur = _relu6(_mm(wi, x2) + bi)
    ur = ur + _relu6(_mm(wm0, ur) + bm0)
    ur = ur + _relu6(_mm(wm1, ur) + bm1)
    ur = _relu6(_mm(wo, ur) + bo)
    return jnp.where(mb > 0.0, ur, x2)


def _prep_body(cp_ref, mask_ref):
    unc = 1.0 - jnp.max(cp_ref[...], axis=0)
    neg = jnp.float32(-1.0)
    up = jnp.concatenate([unc[1:, :], jnp.full((1, W), neg)], axis=0)
    dn = jnp.concatenate([jnp.full((1, W), neg), unc[:-1, :]], axis=0)
    mv = jnp.maximum(jnp.maximum(up, dn), unc)
    lf = jnp.concatenate([mv[:, 1:], jnp.full((H, 1), neg)], axis=1)
    rt = jnp.concatenate([jnp.full((H, 1), neg), mv[:, :-1]], axis=1)
    dil = jnp.maximum(jnp.maximum(lf, rt), mv)
    mask_ref[...] = (dil > GATE).astype(jnp.float32)


def _prep(cp):
    return pl.pallas_call(
        _prep_body,
        out_shape=jax.ShapeDtypeStruct((H, W), jnp.float32),
        in_specs=[pl.BlockSpec((NC, H, W), lambda: (0, 0, 0))],
        out_specs=pl.BlockSpec((H, W), lambda: (0, 0)),
    )(cp)


def _mlp0_body(feat_ref, cp_ref, mask_ref, wi, bi, wm0, bm0, wm1, bm1, wo, bo,
               out_ref):
    j = pl.program_id(0)
    nj = pl.num_programs(0)
    npad = PAD // RS
    interior = (j >= npad) & (j < nj - npad)

    @pl.when(~interior)
    def _():
        out_ref[...] = jnp.zeros_like(out_ref)

    @pl.when(interior)
    def _():
        x = jnp.concatenate([feat_ref[...], cp_ref[...]], axis=0)
        out_ref[...] = x
        mask = mask_ref[...]
        for wc in range(NWC):
            sl = slice(wc * 128, (wc + 1) * 128)
            xm = x[:, :, sl].reshape(C0, RS * 128)
            mb = mask[:, sl].reshape(1, RS * 128)

            @pl.when(jnp.max(mb) > 0.0)
            def _(xm=xm, mb=mb, sl=sl):
                res = _mlp_blend(xm, mb, wi[...], bi[...], wm0[...], bm0[...],
                                 wm1[...], bm1[...], wo[...], bo[...])
                out_ref[:, :, sl] = res.reshape(C0, RS, 128)


def _img_strip8(j):
    return jnp.clip(j - PAD // RS, 0, H // RS - 1)


def _mlp0(feat, cp, mask, ws):
    full = lambda shp: pl.BlockSpec(shp, lambda j: (0,) * len(shp))
    return pl.pallas_call(
        _mlp0_body,
        grid=(NGM,),
        out_shape=jax.ShapeDtypeStruct((C0, HP, W), jnp.float32),
        in_specs=[
            pl.BlockSpec((CIN, RS, W), lambda j: (0, _img_strip8(j), 0)),
            pl.BlockSpec((NC, RS, W), lambda j: (0, _img_strip8(j), 0)),
            pl.BlockSpec((RS, W), lambda j: (_img_strip8(j), 0)),
        ] + [full(w.shape) for w in ws],
        out_specs=pl.BlockSpec((C0, RS, W), lambda j: (0, j, 0)),
        compiler_params=pltpu.CompilerParams(
            dimension_semantics=("arbitrary",)),
    )(feat, cp, mask, *ws)


def _chain_core(g0, up, mid, dn, cw):
    ws = jnp.concatenate([up, mid, dn], axis=1)
    win = ws[:, 5:RSC + 11, :]
    (w90, pw0, sc0, bi0, w91, pw1, sc1, bi1, w9o, pwo, sco, bio) = cw
    y1 = win[:, 1:RSC + 5, :] + _sep(win, w90, pw0, sc0, bi0)
    y1 = _zero_outside(y1, g0 - 2)
    y2 = y1[:, 1:RSC + 3, :] + _sep(y1, w91, pw1, sc1, bi1)
    y2 = _zero_outside(y2, g0 - 1)
    return _sep(y2, w9o, pwo, sco, bio)


def _chain0_body(xu_ref, xc_ref, xd_ref, mask_ref,
                 w90, pw0, sc0, bi0, w91, pw1, sc1, bi1, w9o, pwo, sco, bio,
                 wi, bi1_, wm0, bm0, wm1, bm1, wo, bo, out_ref):
    j = pl.program_id(0)
    nj = pl.num_programs(0)
    interior = (j > 0) & (j < nj - 1)

    @pl.when(~interior)
    def _():
        out_ref[...] = jnp.zeros_like(out_ref)

    @pl.when(interior)
    def _():
        g0 = (j - 1) * RSC
        cw = (w90[...], pw0[...], sc0[...], bi0[...], w91[...], pw1[...],
              sc1[...], bi1[...], w9o[...], pwo[...], sco[...], bio[...])
        o = _chain_core(g0, xu_ref[...], xc_ref[...], xd_ref[...], cw)
        out_ref[...] = o
        mask = mask_ref[...]
        for rc in range(RSC // 8):
            for wc in range(NWC):
                rsl = slice(rc * 8, rc * 8 + 8)
                sl = slice(wc * 128, (wc + 1) * 128)
                xm = o[:, rsl, sl].reshape(O0, 8 * 128)
                mb = mask[rsl, sl].reshape(1, 8 * 128)

                @pl.when(jnp.max(mb) > 0.0)
                def _(xm=xm, mb=mb, rsl=rsl, sl=sl):
                    res = _mlp_blend(xm, mb, wi[...], bi1_[...], wm0[...],
                                     bm0[...], wm1[...], bm1[...], wo[...],
                                     bo[...])
                    out_ref[:, rsl, sl] = res.reshape(O0, 8, 128)


def _chain1_body(xu_ref, xc_ref, xd_ref,
                 w90, pw0, sc0, bi0, w91, pw1, sc1, bi1, w9o, pwo, sco, bio,
                 out_ref):
    j = pl.program_id(0)
    nj = pl.num_programs(0)
    interior = (j > 0) & (j < nj - 1)

    @pl.when(~interior)
    def _():
        out_ref[...] = jnp.zeros_like(out_ref)

    @pl.when(interior)
    def _():
        g0 = (j - 1) * RSC
        cw = (w90[...], pw0[...], sc0[...], bi0[...], w91[...], pw1[...],
              sc1[...], bi1[...], w9o[...], pwo[...], sco[...], bio[...])
        out_ref[...] = _chain_core(g0, xu_ref[...], xc_ref[...], xd_ref[...],
                                   cw)


def _cl8(i):
    return jnp.clip(i, 0, NB8 - 1)


def _halo_specs(c):
    return [
        pl.BlockSpec((c, 8, W), lambda j: (0, _cl8(2 * j - 1), 0)),
        pl.BlockSpec((c, RSC, W), lambda j: (0, j, 0)),
        pl.BlockSpec((c, 8, W), lambda j: (0, _cl8(2 * j + 2), 0)),
    ]


def _mask_strip16(j):
    return jnp.clip(j - 1, 0, H // RSC - 1)


def _chain0(x0p, mask, convw, mlpw):
    full = lambda shp: pl.BlockSpec(shp, lambda j: (0,) * len(shp))
    ws = list(convw) + list(mlpw)
    return pl.pallas_call(
        _chain0_body,
        grid=(NGC,),
        out_shape=jax.ShapeDtypeStruct((O0, HP, W), jnp.float32),
        in_specs=_halo_specs(C0) + [
            pl.BlockSpec((RSC, W), lambda j: (_mask_strip16(j), 0)),
        ] + [full(w.shape) for w in ws],
        out_specs=pl.BlockSpec((O0, RSC, W), lambda j: (0, j, 0)),
        compiler_params=pltpu.CompilerParams(
            dimension_semantics=("arbitrary",)),
    )(x0p, x0p, x0p, mask, *ws)


def _chain1(x1p, convw):
    full = lambda shp: pl.BlockSpec(shp, lambda j: (0,) * len(shp))
    return pl.pallas_call(
        _chain1_body,
        grid=(NGC,),
        out_shape=jax.ShapeDtypeStruct((O1, HP, W), jnp.float32),
        in_specs=_halo_specs(O0) + [full(w.shape) for w in convw],
        out_specs=pl.BlockSpec((O1, RSC, W), lambda j: (0, j, 0)),
        compiler_params=pltpu.CompilerParams(
            dimension_semantics=("arbitrary",)),
    )(x1p, x1p, x1p, *convw)


def _mlp_weights(p, b):
    return (p[f'b{b}_win'], p[f'b{b}_bin'].reshape(-1, 1),
            p[f'b{b}_wm0'], p[f'b{b}_bm0'].reshape(-1, 1),
            p[f'b{b}_wm1'], p[f'b{b}_bm1'].reshape(-1, 1),
            p[f'b{b}_wout'], p[f'b{b}_bout'].reshape(-1, 1))


def _conv_weights(p, b):
    out = []
    scale = np.float32(1.0 / np.sqrt(1.0 + 1e-5))
    for tag in ('0', '1', 'o'):
        dw = p[f'b{b}_dw{tag}']
        pw = p[f'b{b}_pw{tag}']
        g = p[f'b{b}_g{tag}']
        be = p[f'b{b}_be{tag}']
        out += [dw.reshape(dw.shape[0], 9),
                pw.reshape(pw.shape[0], pw.shape[1]),
                (g * scale).reshape(-1, 1), be.reshape(-1, 1)]
    return tuple(out)


def kernel(feature_map, coarse_pred, params):
    feat = feature_map[0]
    cp = coarse_pred[0]
    mask = _prep(cp)
    x0p = _mlp0(feat, cp, mask, _mlp_weights(params, 0))
    x1p = _chain0(x0p, mask, _conv_weights(params, 0), _mlp_weights(params, 1))
    x2p = _chain1(x1p, _conv_weights(params, 1))
    return x2p[None, :, PAD:PAD + H, :]
